# Initial kernel scaffold; baseline (speedup 1.0000x reference)
import jax
import jax.numpy as jnp
from jax.experimental import pallas as pl


def kernel(query, query_points, input, input_points, W_off, b_off, W_attn, b_attn, W_val, b_val, W_out, b_out):
    raise NotImplementedError("write your pallas kernel here")


# speedup vs baseline: 58.8801x; 58.8801x over previous
import functools

import jax
import jax.numpy as jnp
from jax import lax
from jax.experimental import pallas as pl
from jax.experimental.pallas import tpu as pltpu
from jax.experimental.pallas import tpu_sc as plsc

_M = 8
_P = 4

_NC = 2
_NS = 16
_NW = _NC * _NS
_CH = 128


def _prep_body(q_ref, qp_ref, wall_ref, ball_ref, inp_ref, wval_ref,
               bval_ref, samp_ref, attw_ref, val_ref):
    G = _M * _P
    proj = jnp.dot(q_ref[0], wall_ref[...],
                   preferred_element_type=jnp.float32) + ball_ref[...]
    qp_t = jnp.concatenate([qp_ref[0]] * G, axis=1)
    samp_ref[0] = qp_t + proj[:, :3 * G]
    logits = proj[:, 3 * G:3 * G + G]
    sl = [logits[:, i * _M:(i + 1) * _M] for i in range(_P)]
    mx = jnp.maximum(jnp.maximum(sl[0], sl[1]), jnp.maximum(sl[2], sl[3]))
    e = jnp.exp(logits - jnp.concatenate([mx] * _P, axis=1))
    es = [e[:, i * _M:(i + 1) * _M] for i in range(_P)]
    ssum = (es[0] + es[1]) + (es[2] + es[3])
    attw_ref[0] = e / jnp.concatenate([ssum] * _P, axis=1)
    val_ref[0] = jnp.dot(inp_ref[0], wval_ref[...],
                         preferred_element_type=jnp.float32) + bval_ref[...]


def _argmin_body(sp_ref, ipt_ref, ip2_ref, out_ref, *, T, Lin):
    b = pl.program_id(0)
    sp = sp_ref[0]
    ipt = ipt_ref[0]
    ip2 = ip2_ref[0]
    sp2 = jnp.sum(sp * sp, axis=1, keepdims=True)
    d2 = (sp2 + ip2) - 2.0 * jnp.dot(
        sp, ipt, preferred_element_type=jnp.float32)
    mn = jnp.min(d2, axis=1, keepdims=True)
    lane = lax.broadcasted_iota(jnp.int32, d2.shape, 1)
    idx = jnp.min(jnp.where(d2 == mn, lane, Lin), axis=1, keepdims=True)
    r = lax.broadcasted_iota(jnp.int32, (T, 1), 0)
    m = (r // _P) % _M
    out_ref[0] = idx + (b * _M + m) * Lin


def _combine_body(f_ref, a_ref, e_ref, w_ref, b_ref, o_ref):
    a32 = jnp.dot(a_ref[0], e_ref[...],
                  preferred_element_type=jnp.float32)
    o_ref[0] = jnp.dot(f_ref[0] * a32, w_ref[...],
                       preferred_element_type=jnp.float32) + b_ref[...]


def _sc_gather(table, idx3, n_rows, d):
    rows_w = n_rows // _NW
    nch = rows_w // _CH
    mesh = plsc.VectorSubcoreMesh(core_axis_name="c", subcore_axis_name="s")

    @functools.partial(
        pl.kernel,
        mesh=mesh,
        out_type=jax.ShapeDtypeStruct((n_rows, d), jnp.float32),
        scratch_types=[
            pltpu.VMEM((nch, _CH), jnp.int32),
            pltpu.VMEM((rows_w, d), jnp.float32),
            pltpu.SemaphoreType.DMA,
        ],
        compiler_params=pltpu.CompilerParams(use_tc_tiling_on_sc=False),
    )
    def gather_kernel(table_hbm, idx_hbm, out_hbm, idx_v, rows_v, sem):
        wid = lax.axis_index("s") * _NC + lax.axis_index("c")
        pltpu.sync_copy(idx_hbm.at[wid], idx_v)
        copies = []
        for j in range(nch):
            copies.append(pltpu.async_copy(
                table_hbm.at[idx_v.at[j]],
                rows_v.at[pl.ds(j * _CH, _CH)], sem))
        for c in copies:
            c.wait()
        pltpu.sync_copy(rows_v, out_hbm.at[pl.ds(wid * rows_w, rows_w)])

    return gather_kernel(table, idx3)


def kernel(query, query_points, input, input_points, W_off, b_off,
           W_attn, b_attn, W_val, b_val, W_out, b_out):
    B, Lq, C = query.shape
    Lin = input.shape[1]
    M, P = _M, _P
    G = M * P
    D = C // M
    N = Lq * G
    T = 512

    f32 = jnp.float32

    W_attn_r = W_attn.reshape(C, M, P).transpose(0, 2, 1).reshape(C, G)
    b_attn_r = b_attn.reshape(M, P).transpose(1, 0).reshape(G)
    W_all = jnp.concatenate([W_off, W_attn_r], axis=1)
    b_all = jnp.concatenate([b_off, b_attn_r])[None]
    ip_t = input_points.transpose(0, 2, 1)
    ip2 = jnp.sum(input_points * input_points, axis=-1)[:, None, :]
    E = jnp.repeat(jnp.eye(G, dtype=f32), D, axis=1)
    W_out_big = jnp.broadcast_to(
        W_out.reshape(M, 1, D, C), (M, P, D, C)).reshape(G * D, C)
    b_out2 = b_out[None]

    samp, attw_r, value = pl.pallas_call(
        _prep_body,
        grid=(B,),
        in_specs=[
            pl.BlockSpec((1, Lq, C), lambda b: (b, 0, 0)),
            pl.BlockSpec((1, Lq, 3), lambda b: (b, 0, 0)),
            pl.BlockSpec((C, 4 * G), lambda b: (0, 0)),
            pl.BlockSpec((1, 4 * G), lambda b: (0, 0)),
            pl.BlockSpec((1, Lin, C), lambda b: (b, 0, 0)),
            pl.BlockSpec((C, C), lambda b: (0, 0)),
            pl.BlockSpec((1, C), lambda b: (0, 0)),
        ],
        out_specs=[
            pl.BlockSpec((1, Lq, 3 * G), lambda b: (b, 0, 0)),
            pl.BlockSpec((1, Lq, G), lambda b: (b, 0, 0)),
            pl.BlockSpec((1, Lin, C), lambda b: (b, 0, 0)),
        ],
        out_shape=[
            jax.ShapeDtypeStruct((B, Lq, 3 * G), f32),
            jax.ShapeDtypeStruct((B, Lq, G), f32),
            jax.ShapeDtypeStruct((B, Lin, C), f32),
        ],
    )(query, query_points, W_all, b_all, input, W_val, b_val[None])

    sp_flat = samp.reshape(B, N, 3)
    gidx = pl.pallas_call(
        functools.partial(_argmin_body, T=T, Lin=Lin),
        grid=(B, N // T),
        in_specs=[
            pl.BlockSpec((1, T, 3), lambda b, t: (b, t, 0)),
            pl.BlockSpec((1, 3, Lin), lambda b, t: (b, 0, 0)),
            pl.BlockSpec((1, 1, Lin), lambda b, t: (b, 0, 0)),
        ],
        out_specs=pl.BlockSpec((1, T, 1), lambda b, t: (b, t, 0)),
        out_shape=jax.ShapeDtypeStruct((B, N, 1), jnp.int32),
    )(sp_flat, ip_t, ip2)

    value_t = value.reshape(B, Lin, M, D).transpose(0, 2, 1, 3)
    value_t = value_t.reshape(B * M * Lin, D)
    n_rows = B * N
    idx3 = gidx.reshape(_NW, n_rows // _NW // _CH, _CH)
    feat = _sc_gather(value_t, idx3, n_rows, D)

    F = feat.reshape(B, Lq, G * D)
    attw_m = attw_r.reshape(B, Lq, P, M).transpose(0, 1, 3, 2)
    attw_m = attw_m.reshape(B, Lq, G)
    out = pl.pallas_call(
        _combine_body,
        grid=(B,),
        in_specs=[
            pl.BlockSpec((1, Lq, G * D), lambda b: (b, 0, 0)),
            pl.BlockSpec((1, Lq, G), lambda b: (b, 0, 0)),
            pl.BlockSpec((G, G * D), lambda b: (0, 0)),
            pl.BlockSpec((G * D, C), lambda b: (0, 0)),
            pl.BlockSpec((1, C), lambda b: (0, 0)),
        ],
        out_specs=pl.BlockSpec((1, Lq, C), lambda b: (b, 0, 0)),
        out_shape=jax.ShapeDtypeStruct((B, Lq, C), f32),
    )(F, attw_m, E, W_out_big, b_out2)
    return out


# speedup vs baseline: 65.2556x; 1.1083x over previous
import functools

import jax
import jax.numpy as jnp
from jax import lax
from jax.experimental import pallas as pl
from jax.experimental.pallas import tpu as pltpu
from jax.experimental.pallas import tpu_sc as plsc

_M = 8
_P = 4

_NC = 2
_NS = 16
_NW = _NC * _NS
_CH = 128


def _prep_body(q_ref, qp_ref, wall_ref, ball_ref, inp_ref, wval_ref,
               bval_ref, samp_ref, attw_ref, val_ref):
    G = _M * _P
    proj = jnp.dot(q_ref[0], wall_ref[...],
                   preferred_element_type=jnp.float32) + ball_ref[...]
    qp_t = jnp.concatenate([qp_ref[0]] * G, axis=1)
    samp_ref[0] = qp_t + proj[:, :3 * G]
    logits = proj[:, 3 * G:3 * G + G]
    sl = [logits[:, i * _M:(i + 1) * _M] for i in range(_P)]
    mx = jnp.maximum(jnp.maximum(sl[0], sl[1]), jnp.maximum(sl[2], sl[3]))
    e = jnp.exp(logits - jnp.concatenate([mx] * _P, axis=1))
    es = [e[:, i * _M:(i + 1) * _M] for i in range(_P)]
    ssum = (es[0] + es[1]) + (es[2] + es[3])
    attw_ref[0] = e / jnp.concatenate([ssum] * _P, axis=1)
    val_ref[0] = jnp.dot(inp_ref[0], wval_ref[...],
                         preferred_element_type=jnp.float32) + bval_ref[...]


def _argmin_body(sp_ref, ipt_ref, ip2_ref, out_ref, *, T, Lin):
    b = pl.program_id(0)
    sp = sp_ref[0]
    ipt = ipt_ref[0]
    ip2 = ip2_ref[0]
    sp2 = jnp.sum(sp * sp, axis=1, keepdims=True)
    d2 = (sp2 + ip2) - 2.0 * jnp.dot(
        sp, ipt, preferred_element_type=jnp.float32)
    mn = jnp.min(d2, axis=1, keepdims=True)
    lane = lax.broadcasted_iota(jnp.int32, d2.shape, 1)
    idx = jnp.min(jnp.where(d2 == mn, lane, Lin), axis=1, keepdims=True)
    r = lax.broadcasted_iota(jnp.int32, (T, 1), 0)
    m = (r // _P) % _M
    out_ref[0] = (idx + b * Lin) * _M + m


def _combine_body(f_ref, a_ref, e_ref, w_ref, b_ref, o_ref):
    a32 = jnp.dot(a_ref[0], e_ref[...],
                  preferred_element_type=jnp.float32)
    o_ref[0] = jnp.dot(f_ref[0] * a32, w_ref[...],
                       preferred_element_type=jnp.float32) + b_ref[...]


def _sc_gather(table, idx3, n_rows, d):
    rows_w = n_rows // _NW
    nch = rows_w // _CH
    mesh = plsc.VectorSubcoreMesh(core_axis_name="c", subcore_axis_name="s")

    @functools.partial(
        pl.kernel,
        mesh=mesh,
        out_type=jax.ShapeDtypeStruct((n_rows, d), jnp.float32),
        scratch_types=[
            pltpu.VMEM((nch, _CH), jnp.int32),
            pltpu.VMEM((rows_w, d), jnp.float32),
            pltpu.SemaphoreType.DMA,
        ],
        compiler_params=pltpu.CompilerParams(use_tc_tiling_on_sc=False),
    )
    def gather_kernel(table_hbm, idx_hbm, out_hbm, idx_v, rows_v, sem):
        wid = lax.axis_index("s") * _NC + lax.axis_index("c")
        pltpu.sync_copy(idx_hbm.at[wid], idx_v)
        copies = []
        for j in range(nch):
            copies.append(pltpu.async_copy(
                table_hbm.at[idx_v.at[j]],
                rows_v.at[pl.ds(j * _CH, _CH)], sem))
        for c in copies:
            c.wait()
        pltpu.sync_copy(rows_v, out_hbm.at[pl.ds(wid * rows_w, rows_w)])

    return gather_kernel(table, idx3)


def kernel(query, query_points, input, input_points, W_off, b_off,
           W_attn, b_attn, W_val, b_val, W_out, b_out):
    B, Lq, C = query.shape
    Lin = input.shape[1]
    M, P = _M, _P
    G = M * P
    D = C // M
    N = Lq * G
    T = 512

    f32 = jnp.float32

    W_attn_r = W_attn.reshape(C, M, P).transpose(0, 2, 1).reshape(C, G)
    b_attn_r = b_attn.reshape(M, P).transpose(1, 0).reshape(G)
    W_all = jnp.concatenate([W_off, W_attn_r], axis=1)
    b_all = jnp.concatenate([b_off, b_attn_r])[None]
    ip_t = input_points.transpose(0, 2, 1)
    ip2 = jnp.sum(input_points * input_points, axis=-1)[:, None, :]
    perm = (jnp.arange(G) % _M) * _P + jnp.arange(G) // _M
    E = jnp.repeat(jnp.eye(G, dtype=f32)[perm], D, axis=1)
    W_out_big = jnp.broadcast_to(
        W_out.reshape(M, 1, D, C), (M, P, D, C)).reshape(G * D, C)
    b_out2 = b_out[None]

    samp, attw_r, value = pl.pallas_call(
        _prep_body,
        grid=(B,),
        in_specs=[
            pl.BlockSpec((1, Lq, C), lambda b: (b, 0, 0)),
            pl.BlockSpec((1, Lq, 3), lambda b: (b, 0, 0)),
            pl.BlockSpec((C, 4 * G), lambda b: (0, 0)),
            pl.BlockSpec((1, 4 * G), lambda b: (0, 0)),
            pl.BlockSpec((1, Lin, C), lambda b: (b, 0, 0)),
            pl.BlockSpec((C, C), lambda b: (0, 0)),
            pl.BlockSpec((1, C), lambda b: (0, 0)),
        ],
        out_specs=[
            pl.BlockSpec((1, Lq, 3 * G), lambda b: (b, 0, 0)),
            pl.BlockSpec((1, Lq, G), lambda b: (b, 0, 0)),
            pl.BlockSpec((1, Lin, C), lambda b: (b, 0, 0)),
        ],
        out_shape=[
            jax.ShapeDtypeStruct((B, Lq, 3 * G), f32),
            jax.ShapeDtypeStruct((B, Lq, G), f32),
            jax.ShapeDtypeStruct((B, Lin, C), f32),
        ],
    )(query, query_points, W_all, b_all, input, W_val, b_val[None])

    sp_flat = samp.reshape(B, N, 3)
    gidx = pl.pallas_call(
        functools.partial(_argmin_body, T=T, Lin=Lin),
        grid=(B, N // T),
        in_specs=[
            pl.BlockSpec((1, T, 3), lambda b, t: (b, t, 0)),
            pl.BlockSpec((1, 3, Lin), lambda b, t: (b, 0, 0)),
            pl.BlockSpec((1, 1, Lin), lambda b, t: (b, 0, 0)),
        ],
        out_specs=pl.BlockSpec((1, T, 1), lambda b, t: (b, t, 0)),
        out_shape=jax.ShapeDtypeStruct((B, N, 1), jnp.int32),
    )(sp_flat, ip_t, ip2)

    value_t = value.reshape(B * Lin * M, D)
    n_rows = B * N
    idx3 = gidx.reshape(_NW, n_rows // _NW // _CH, _CH)
    feat = _sc_gather(value_t, idx3, n_rows, D)

    F = feat.reshape(B, Lq, G * D)
    out = pl.pallas_call(
        _combine_body,
        grid=(B,),
        in_specs=[
            pl.BlockSpec((1, Lq, G * D), lambda b: (b, 0, 0)),
            pl.BlockSpec((1, Lq, G), lambda b: (b, 0, 0)),
            pl.BlockSpec((G, G * D), lambda b: (0, 0)),
            pl.BlockSpec((G * D, C), lambda b: (0, 0)),
            pl.BlockSpec((1, C), lambda b: (0, 0)),
        ],
        out_specs=pl.BlockSpec((1, Lq, C), lambda b: (b, 0, 0)),
        out_shape=jax.ShapeDtypeStruct((B, Lq, C), f32),
    )(F, attw_r, E, W_out_big, b_out2)
    return out
